# baseline (speedup 1.0000x reference)
import jax
import jax.numpy as jnp
from jax.experimental import pallas as pl
from jax.experimental.pallas import tpu as pltpu

_B, _H, _W, _C = 2, 384, 384, 96
_RB = 16


def _unpool_mask_body(x_ref, o_ref):
    xb = x_ref[0]
    rb, w, c = xb.shape
    x4 = xb.reshape(rb // 2, 2, w, c)
    xe = x4[:, 0]
    xo = x4[:, 1]

    even_w = (jax.lax.broadcasted_iota(jnp.int32, xe.shape, 1) & 1) == 0

    def pair_swap_w(a):
        return jnp.where(
            even_w,
            pltpu.roll(a, w - 1, axis=1),
            pltpu.roll(a, 1, axis=1),
        )

    pw_e = pair_swap_w(xe)
    pw_o = pair_swap_w(xo)

    m = jnp.maximum(jnp.maximum(xe, pw_e), jnp.maximum(xo, pw_o))

    ee = xe == m
    eo = xo == m
    epe = pw_e == m
    epo = pw_o == m

    surv_e = ee & (even_w | ~epe)
    surv_o = eo & ~ee & ~epe & (even_w | ~epo)

    oe = jnp.where(surv_e, xe, 0.0)
    oo = jnp.where(surv_o, xo, 0.0)

    o_ref[0] = jnp.stack([oe, oo], axis=1).reshape(rb, w, c)


def kernel(x):
    grid = (_B, _H // _RB)
    return pl.pallas_call(
        _unpool_mask_body,
        grid=grid,
        in_specs=[pl.BlockSpec((1, _RB, _W, _C), lambda b, i: (b, i, 0, 0))],
        out_specs=pl.BlockSpec((1, _RB, _W, _C), lambda b, i: (b, i, 0, 0)),
        out_shape=jax.ShapeDtypeStruct((_B, _H, _W, _C), x.dtype),
    )(x)


# speedup vs baseline: 2.7714x; 2.7714x over previous
import jax
import jax.numpy as jnp
from jax.experimental import pallas as pl
from jax.experimental.pallas import tpu as pltpu

_B, _H, _W, _C = 2, 384, 384, 96
_RB = 16


def _unpool_mask_body(x_ref, o_ref):
    xb = x_ref[0]
    rb, c, w = xb.shape
    x4 = xb.reshape(rb // 2, 2, c, w)
    xe = x4[:, 0]
    xo = x4[:, 1]

    even_w = (jax.lax.broadcasted_iota(jnp.int32, xe.shape, 2) & 1) == 0

    def pair_swap_w(a):
        return jnp.where(
            even_w,
            pltpu.roll(a, w - 1, axis=2),
            pltpu.roll(a, 1, axis=2),
        )

    pw_e = pair_swap_w(xe)
    pw_o = pair_swap_w(xo)

    m = jnp.maximum(jnp.maximum(xe, pw_e), jnp.maximum(xo, pw_o))

    ee = xe == m
    eo = xo == m
    epe = pw_e == m
    epo = pw_o == m

    surv_e = ee & (even_w | ~epe)
    surv_o = eo & ~ee & ~epe & (even_w | ~epo)

    oe = jnp.where(surv_e, xe, 0.0)
    oo = jnp.where(surv_o, xo, 0.0)

    o_ref[0] = jnp.stack([oe, oo], axis=1).reshape(rb, c, w)


def kernel(x):
    xt = jnp.transpose(x, (0, 1, 3, 2))
    out = pl.pallas_call(
        _unpool_mask_body,
        grid=(_B, _H // _RB),
        in_specs=[pl.BlockSpec((1, _RB, _C, _W), lambda b, i: (b, i, 0, 0))],
        out_specs=pl.BlockSpec((1, _RB, _C, _W), lambda b, i: (b, i, 0, 0)),
        out_shape=jax.ShapeDtypeStruct((_B, _H, _C, _W), x.dtype),
    )(xt)
    return jnp.transpose(out, (0, 1, 3, 2))
